# baseline (speedup 1.0000x reference)
import functools

import jax
import jax.numpy as jnp
from jax import lax
from jax.experimental import pallas as pl
from jax.experimental.pallas import tpu as pltpu
from jax.experimental.pallas import tpu_sc as plsc

N_NODES = 10000
N_EDGES = 320000
DIM = 128
DEGW = 16
NPAD = 10016
CHUNK = 64
EROWS = N_EDGES // CHUNK
NCH = 157
ROWS_PER_TILE = NPAD // 16
BM = 1000

_mesh = plsc.VectorSubcoreMesh(core_axis_name="c", subcore_axis_name="s")


@functools.partial(
    pl.kernel,
    out_type=[
        jax.ShapeDtypeStruct((NPAD, DIM), jnp.float32),
        jax.ShapeDtypeStruct((NPAD, DIM), jnp.float32),
        jax.ShapeDtypeStruct((NPAD, DEGW), jnp.float32),
        jax.ShapeDtypeStruct((NPAD, DEGW), jnp.float32),
    ],
    mesh=_mesh,
    compiler_params=pltpu.CompilerParams(use_tc_tiling_on_sc=False),
    scratch_types=[
        pltpu.VMEM_SHARED((NPAD, DIM), jnp.float32),
        pltpu.VMEM_SHARED((NPAD, DEGW), jnp.float32),
        pltpu.VMEM((NCH, CHUNK), jnp.int32),
        pltpu.VMEM((NCH, CHUNK), jnp.int32),
        pltpu.VMEM((CHUNK, DIM), jnp.float32),
        pltpu.VMEM((CHUNK, DIM), jnp.float32),
        pltpu.VMEM((CHUNK, DEGW), jnp.float32),
        pltpu.SemaphoreType.DMA,
        pltpu.SemaphoreType.DMA,
        pltpu.SemaphoreType.DMA,
        pltpu.SemaphoreType.DMA,
        pltpu.SemaphoreType.DMA,
        pltpu.SemaphoreType.DMA,
    ],
)
def _segsum_sc(x, ei, zh, zd, h0, h1, d0, d1, acch, accd, sidx, didx, b0, b1,
               ones, sg0, sg1, sa0, sa1, sd0, sd1):
    c = lax.axis_index("c")
    s = lax.axis_index("s")
    wid = s * 2 + c
    my_base = s * ROWS_PER_TILE
    myrows = pl.ds(my_base, ROWS_PER_TILE)

    base_raw = wid * NCH
    nch = jnp.minimum(NCH, EROWS - base_raw)
    base = jnp.minimum(base_raw, EROWS - NCH)
    skew = base_raw - base

    pltpu.async_copy(ei.at[0].at[pl.ds(base, NCH)], sidx, sg0)
    pltpu.async_copy(ei.at[1].at[pl.ds(base, NCH)], didx, sg1)
    pltpu.async_copy(zh.at[myrows], acch.at[myrows], sa0)
    pltpu.async_copy(zd.at[myrows], accd.at[myrows], sd0)

    ones16 = jnp.ones((16,), jnp.float32)

    def fillones(r, carry):
        ones[r, :] = ones16
        return carry

    lax.fori_loop(0, CHUNK, fillones, 0)

    pltpu.make_async_copy(zh.at[myrows], acch.at[myrows], sa0).wait()
    pltpu.make_async_copy(zd.at[myrows], accd.at[myrows], sd0).wait()
    plsc.subcore_barrier()
    pltpu.make_async_copy(ei.at[0].at[pl.ds(base, NCH)], sidx, sg0).wait()
    pltpu.make_async_copy(ei.at[1].at[pl.ds(base, NCH)], didx, sg1).wait()

    def gat(k, buf, sem):
        pltpu.async_copy(x.at[sidx.at[skew + k]], buf, sem)

    def gat_wait(k, buf, sem):
        pltpu.make_async_copy(x.at[sidx.at[skew + k]], buf, sem).wait()

    def adds(k, buf, sema, semd):
        pltpu.async_copy(buf, acch.at[didx.at[skew + k]], sema, add=True)
        pltpu.async_copy(ones, accd.at[didx.at[skew + k]], semd, add=True)

    def adds_wait(k, buf, sema, semd):
        pltpu.make_async_copy(buf, acch.at[didx.at[skew + k]], sema).wait()
        pltpu.make_async_copy(ones, accd.at[didx.at[skew + k]], semd).wait()

    gat(0, b0, sg0)
    gat(1, b1, sg1)

    def body(i, carry):
        k = 2 * i
        gat_wait(k, b0, sg0)
        adds(k, b0, sa0, sd0)
        gat_wait(k + 1, b1, sg1)
        adds(k + 1, b1, sa1, sd1)
        adds_wait(k, b0, sa0, sd0)

        @pl.when(k + 2 < nch)
        def _():
            gat(k + 2, b0, sg0)

        adds_wait(k + 1, b1, sa1, sd1)

        @pl.when(k + 3 < nch)
        def _():
            gat(k + 3, b1, sg1)

        return carry

    lax.fori_loop(0, nch // 2, body, 0)

    @pl.when(nch % 2 == 1)
    def _():
        last = nch - 1
        gat_wait(last, b0, sg0)
        adds(last, b0, sa0, sd0)
        adds_wait(last, b0, sa0, sd0)

    plsc.subcore_barrier()

    @pl.when(c == 0)
    def _():
        pltpu.async_copy(acch.at[myrows], h0.at[myrows], sa0)
        pltpu.async_copy(accd.at[myrows], d0.at[myrows], sd0)
        pltpu.make_async_copy(acch.at[myrows], h0.at[myrows], sa0).wait()
        pltpu.make_async_copy(accd.at[myrows], d0.at[myrows], sd0).wait()

    @pl.when(c == 1)
    def _():
        pltpu.async_copy(acch.at[myrows], h1.at[myrows], sa0)
        pltpu.async_copy(accd.at[myrows], d1.at[myrows], sd0)
        pltpu.make_async_copy(acch.at[myrows], h1.at[myrows], sa0).wait()
        pltpu.make_async_copy(accd.at[myrows], d1.at[myrows], sd0).wait()


def _combine_body(p0_ref, p1_ref, d0_ref, d1_ref, x_ref, w_ref, b_ref,
                  f_ref, ssq_ref):
    hsum = p0_ref[...] + p1_ref[...]
    deg = jnp.sum(d0_ref[...] + d1_ref[...], axis=1, keepdims=True) * (1.0 / DEGW)
    deg = jnp.maximum(deg, 1.0)
    h = hsum / deg
    w13 = w_ref[:DIM, :] + w_ref[2 * DIM:, :]
    w2 = w_ref[DIM:2 * DIM, :]
    z = jnp.dot(x_ref[...], w13, preferred_element_type=jnp.float32,
                precision=lax.Precision.HIGHEST)
    z = z + jnp.dot(h, w2, preferred_element_type=jnp.float32,
                    precision=lax.Precision.HIGHEST)
    f = jnp.tanh(z + b_ref[...])
    f_ref[...] = f

    blk = jnp.sum(f * f)[None, None]

    @pl.when(pl.program_id(0) == 0)
    def _():
        ssq_ref[...] = blk

    @pl.when(pl.program_id(0) > 0)
    def _():
        ssq_ref[...] = ssq_ref[...] + blk


def _scale_body(f_ref, ssq_ref, o_ref):
    o_ref[...] = f_ref[...] * lax.rsqrt(ssq_ref[...])


def kernel(x, edge_index, W, b):
    x = x.astype(jnp.float32)
    ei = edge_index.astype(jnp.int32).reshape(2, EROWS, CHUNK)

    zh = jnp.zeros((NPAD, DIM), jnp.float32)
    zd = jnp.zeros((NPAD, DEGW), jnp.float32)
    p0, p1, d0, d1 = _segsum_sc(x, ei, zh, zd)

    grid = N_NODES // BM
    f, ssq = pl.pallas_call(
        _combine_body,
        grid=(grid,),
        in_specs=[
            pl.BlockSpec((BM, DIM), lambda i: (i, 0)),
            pl.BlockSpec((BM, DIM), lambda i: (i, 0)),
            pl.BlockSpec((BM, DEGW), lambda i: (i, 0)),
            pl.BlockSpec((BM, DEGW), lambda i: (i, 0)),
            pl.BlockSpec((BM, DIM), lambda i: (i, 0)),
            pl.BlockSpec((3 * DIM, DIM), lambda i: (0, 0)),
            pl.BlockSpec((1, DIM), lambda i: (0, 0)),
        ],
        out_specs=[
            pl.BlockSpec((BM, DIM), lambda i: (i, 0)),
            pl.BlockSpec((1, 1), lambda i: (0, 0)),
        ],
        out_shape=[
            jax.ShapeDtypeStruct((N_NODES, DIM), jnp.float32),
            jax.ShapeDtypeStruct((1, 1), jnp.float32),
        ],
    )(p0, p1, d0, d1, x, W, b.reshape(1, DIM))

    out = pl.pallas_call(
        _scale_body,
        grid=(grid,),
        in_specs=[
            pl.BlockSpec((BM, DIM), lambda i: (i, 0)),
            pl.BlockSpec((1, 1), lambda i: (0, 0)),
        ],
        out_specs=pl.BlockSpec((BM, DIM), lambda i: (i, 0)),
        out_shape=jax.ShapeDtypeStruct((N_NODES, DIM), jnp.float32),
    )(f, ssq)
    return out


# speedup vs baseline: 1.0911x; 1.0911x over previous
import functools

import jax
import jax.numpy as jnp
from jax import lax
from jax.experimental import pallas as pl
from jax.experimental.pallas import tpu as pltpu
from jax.experimental.pallas import tpu_sc as plsc

N_NODES = 10000
N_EDGES = 320000
DIM = 128
DEGW = 16
NPAD = 10016
CHUNK = 64
EROWS = N_EDGES // CHUNK
NCH = 157
ROWS_PER_TILE = NPAD // 16
BM = 1000

_mesh = plsc.VectorSubcoreMesh(core_axis_name="c", subcore_axis_name="s")


@functools.partial(
    pl.kernel,
    out_type=[
        jax.ShapeDtypeStruct((NPAD, DIM), jnp.float32),
        jax.ShapeDtypeStruct((NPAD, DIM), jnp.float32),
        jax.ShapeDtypeStruct((NPAD, DEGW), jnp.float32),
        jax.ShapeDtypeStruct((NPAD, DEGW), jnp.float32),
    ],
    mesh=_mesh,
    compiler_params=pltpu.CompilerParams(use_tc_tiling_on_sc=False),
    scratch_types=[
        pltpu.VMEM_SHARED((NPAD, DIM), jnp.float32),
        pltpu.VMEM_SHARED((NPAD, DEGW), jnp.float32),
        pltpu.VMEM((NCH, CHUNK), jnp.int32),
        pltpu.VMEM((NCH, CHUNK), jnp.int32),
        pltpu.VMEM((CHUNK, DIM), jnp.float32),
        pltpu.VMEM((CHUNK, DIM), jnp.float32),
        pltpu.VMEM((CHUNK, DEGW), jnp.float32),
        pltpu.VMEM((CHUNK,), jnp.int32),
        pltpu.SemaphoreType.DMA,
        pltpu.SemaphoreType.DMA,
        pltpu.SemaphoreType.DMA,
        pltpu.SemaphoreType.DMA,
    ],
)
def _segsum_sc(x, ei, zh, zd, h0, h1, d0, d1, acch, accd, sidx, didx, b0, b1,
               ones, jidx, sg0, sg1, sa0, sd0):
    c = lax.axis_index("c")
    s = lax.axis_index("s")
    wid = s * 2 + c
    my_base = s * ROWS_PER_TILE
    myrows = pl.ds(my_base, ROWS_PER_TILE)

    base_raw = wid * NCH
    nch = jnp.minimum(NCH, EROWS - base_raw)
    base = jnp.minimum(base_raw, EROWS - NCH)
    skew = base_raw - base

    pltpu.async_copy(ei.at[0].at[pl.ds(base, NCH)], sidx, sg0)
    pltpu.async_copy(ei.at[1].at[pl.ds(base, NCH)], didx, sg1)
    pltpu.async_copy(zh.at[myrows], acch.at[myrows], sa0)
    pltpu.async_copy(zd.at[myrows], accd.at[myrows], sd0)

    ones16 = jnp.ones((16,), jnp.float32)

    def fillones(r, carry):
        ones[r, :] = ones16
        return carry

    lax.fori_loop(0, CHUNK, fillones, 0)

    pltpu.make_async_copy(zh.at[myrows], acch.at[myrows], sa0).wait()
    pltpu.make_async_copy(zd.at[myrows], accd.at[myrows], sd0).wait()
    plsc.subcore_barrier()
    pltpu.make_async_copy(ei.at[0].at[pl.ds(base, NCH)], sidx, sg0).wait()
    pltpu.make_async_copy(ei.at[1].at[pl.ds(base, NCH)], didx, sg1).wait()

    def gat(k, buf, sem):
        pltpu.async_copy(x.at[sidx.at[skew + k]], buf, sem)

    def gat_wait(k, buf, sem):
        pltpu.make_async_copy(x.at[sidx.at[skew + k]], buf, sem).wait()

    def deg_wait():
        pltpu.make_async_copy(ones, accd.at[jidx], sd0).wait()

    iota16 = lax.iota(jnp.int32, 16)
    for cc in range(CHUNK // 16):
        jidx[pl.ds(cc * 16, 16)] = N_NODES + iota16

    pltpu.async_copy(ones, accd.at[jidx], sd0, add=True)

    gat(0, b0, sg0)
    gat(1, b1, sg1)

    def body(i, carry):
        k = 2 * i
        gat_wait(k, b0, sg0)
        pltpu.sync_copy(b0, acch.at[didx.at[skew + k]], add=True)
        deg_wait()
        pltpu.async_copy(ones, accd.at[didx.at[skew + k]], sd0, add=True)

        @pl.when(k + 2 < nch)
        def _():
            gat(k + 2, b0, sg0)

        gat_wait(k + 1, b1, sg1)
        pltpu.sync_copy(b1, acch.at[didx.at[skew + k + 1]], add=True)
        deg_wait()
        pltpu.async_copy(ones, accd.at[didx.at[skew + k + 1]], sd0, add=True)

        @pl.when(k + 3 < nch)
        def _():
            gat(k + 3, b1, sg1)

        return carry

    lax.fori_loop(0, nch // 2, body, 0)

    @pl.when(nch % 2 == 1)
    def _():
        last = nch - 1
        gat_wait(last, b0, sg0)
        pltpu.sync_copy(b0, acch.at[didx.at[skew + last]], add=True)
        deg_wait()
        pltpu.async_copy(ones, accd.at[didx.at[skew + last]], sd0, add=True)

    deg_wait()
    plsc.subcore_barrier()

    @pl.when(c == 0)
    def _():
        pltpu.async_copy(acch.at[myrows], h0.at[myrows], sa0)
        pltpu.async_copy(accd.at[myrows], d0.at[myrows], sd0)
        pltpu.make_async_copy(acch.at[myrows], h0.at[myrows], sa0).wait()
        pltpu.make_async_copy(accd.at[myrows], d0.at[myrows], sd0).wait()

    @pl.when(c == 1)
    def _():
        pltpu.async_copy(acch.at[myrows], h1.at[myrows], sa0)
        pltpu.async_copy(accd.at[myrows], d1.at[myrows], sd0)
        pltpu.make_async_copy(acch.at[myrows], h1.at[myrows], sa0).wait()
        pltpu.make_async_copy(accd.at[myrows], d1.at[myrows], sd0).wait()


def _combine_body(p0_ref, p1_ref, d0_ref, d1_ref, x_ref, w_ref, b_ref,
                  f_ref, ssq_ref):
    hsum = p0_ref[...] + p1_ref[...]
    deg = jnp.sum(d0_ref[...] + d1_ref[...], axis=1, keepdims=True) * (1.0 / DEGW)
    deg = jnp.maximum(deg, 1.0)
    h = hsum / deg
    w13 = w_ref[:DIM, :] + w_ref[2 * DIM:, :]
    w2 = w_ref[DIM:2 * DIM, :]
    z = jnp.dot(x_ref[...], w13, preferred_element_type=jnp.float32,
                precision=lax.Precision.HIGHEST)
    z = z + jnp.dot(h, w2, preferred_element_type=jnp.float32,
                    precision=lax.Precision.HIGHEST)
    f = jnp.tanh(z + b_ref[...])
    f_ref[...] = f

    blk = jnp.sum(f * f)[None, None]

    @pl.when(pl.program_id(0) == 0)
    def _():
        ssq_ref[...] = blk

    @pl.when(pl.program_id(0) > 0)
    def _():
        ssq_ref[...] = ssq_ref[...] + blk


def _scale_body(f_ref, ssq_ref, o_ref):
    o_ref[...] = f_ref[...] * lax.rsqrt(ssq_ref[...])


def kernel(x, edge_index, W, b):
    x = x.astype(jnp.float32)
    ei = edge_index.astype(jnp.int32).reshape(2, EROWS, CHUNK)

    zh = jnp.zeros((NPAD, DIM), jnp.float32)
    zd = jnp.zeros((NPAD, DEGW), jnp.float32)
    p0, p1, d0, d1 = _segsum_sc(x, ei, zh, zd)

    grid = N_NODES // BM
    f, ssq = pl.pallas_call(
        _combine_body,
        grid=(grid,),
        in_specs=[
            pl.BlockSpec((BM, DIM), lambda i: (i, 0)),
            pl.BlockSpec((BM, DIM), lambda i: (i, 0)),
            pl.BlockSpec((BM, DEGW), lambda i: (i, 0)),
            pl.BlockSpec((BM, DEGW), lambda i: (i, 0)),
            pl.BlockSpec((BM, DIM), lambda i: (i, 0)),
            pl.BlockSpec((3 * DIM, DIM), lambda i: (0, 0)),
            pl.BlockSpec((1, DIM), lambda i: (0, 0)),
        ],
        out_specs=[
            pl.BlockSpec((BM, DIM), lambda i: (i, 0)),
            pl.BlockSpec((1, 1), lambda i: (0, 0)),
        ],
        out_shape=[
            jax.ShapeDtypeStruct((N_NODES, DIM), jnp.float32),
            jax.ShapeDtypeStruct((1, 1), jnp.float32),
        ],
    )(p0, p1, d0, d1, x, W, b.reshape(1, DIM))

    out = pl.pallas_call(
        _scale_body,
        grid=(grid,),
        in_specs=[
            pl.BlockSpec((BM, DIM), lambda i: (i, 0)),
            pl.BlockSpec((1, 1), lambda i: (0, 0)),
        ],
        out_specs=pl.BlockSpec((BM, DIM), lambda i: (i, 0)),
        out_shape=jax.ShapeDtypeStruct((N_NODES, DIM), jnp.float32),
    )(f, ssq)
    return out


# speedup vs baseline: 1.1459x; 1.0502x over previous
import functools

import jax
import jax.numpy as jnp
from jax import lax
from jax.experimental import pallas as pl
from jax.experimental.pallas import tpu as pltpu
from jax.experimental.pallas import tpu_sc as plsc

N_NODES = 10000
N_EDGES = 320000
DIM = 128
DEGW = 16
NPAD = 10016
CHUNK = 64
EROWS = N_EDGES // CHUNK
NCH = 157
ROWS_PER_TILE = NPAD // 16
BM = 2000

_mesh = plsc.VectorSubcoreMesh(core_axis_name="c", subcore_axis_name="s")


@functools.partial(
    pl.kernel,
    out_type=[
        jax.ShapeDtypeStruct((NPAD, DIM), jnp.float32),
        jax.ShapeDtypeStruct((NPAD, DIM), jnp.float32),
        jax.ShapeDtypeStruct((NPAD, DEGW), jnp.float32),
        jax.ShapeDtypeStruct((NPAD, DEGW), jnp.float32),
    ],
    mesh=_mesh,
    compiler_params=pltpu.CompilerParams(use_tc_tiling_on_sc=False),
    scratch_types=[
        pltpu.VMEM_SHARED((NPAD, DIM), jnp.float32),
        pltpu.VMEM_SHARED((NPAD, DEGW), jnp.float32),
        pltpu.VMEM((NCH, CHUNK), jnp.int32),
        pltpu.VMEM((NCH, CHUNK), jnp.int32),
        pltpu.VMEM((CHUNK, DIM), jnp.float32),
        pltpu.VMEM((CHUNK, DIM), jnp.float32),
        pltpu.VMEM((CHUNK, DEGW), jnp.float32),
        pltpu.VMEM((CHUNK,), jnp.int32),
        pltpu.SemaphoreType.DMA,
        pltpu.SemaphoreType.DMA,
        pltpu.SemaphoreType.DMA,
        pltpu.SemaphoreType.DMA,
    ],
)
def _segsum_sc(x, ei, zh, zd, h0, h1, d0, d1, acch, accd, sidx, didx, b0, b1,
               ones, jidx, sg0, sg1, sa0, sd0):
    c = lax.axis_index("c")
    s = lax.axis_index("s")
    wid = s * 2 + c
    my_base = s * ROWS_PER_TILE
    myrows = pl.ds(my_base, ROWS_PER_TILE)

    base_raw = wid * NCH
    nch = jnp.minimum(NCH, EROWS - base_raw)
    base = jnp.minimum(base_raw, EROWS - NCH)
    skew = base_raw - base

    pltpu.async_copy(ei.at[0].at[pl.ds(base, NCH)], sidx, sg0)
    pltpu.async_copy(ei.at[1].at[pl.ds(base, NCH)], didx, sg1)
    pltpu.async_copy(zh.at[myrows], acch.at[myrows], sa0)
    pltpu.async_copy(zd.at[myrows], accd.at[myrows], sd0)

    ones16 = jnp.ones((16,), jnp.float32)

    def fillones(r, carry):
        ones[r, :] = ones16
        return carry

    lax.fori_loop(0, CHUNK, fillones, 0)

    pltpu.make_async_copy(zh.at[myrows], acch.at[myrows], sa0).wait()
    pltpu.make_async_copy(zd.at[myrows], accd.at[myrows], sd0).wait()
    plsc.subcore_barrier()
    pltpu.make_async_copy(ei.at[0].at[pl.ds(base, NCH)], sidx, sg0).wait()
    pltpu.make_async_copy(ei.at[1].at[pl.ds(base, NCH)], didx, sg1).wait()

    def gat(k, buf, sem):
        pltpu.async_copy(x.at[sidx.at[skew + k]], buf, sem)

    def gat_wait(k, buf, sem):
        pltpu.make_async_copy(x.at[sidx.at[skew + k]], buf, sem).wait()

    def deg_wait():
        pltpu.make_async_copy(ones, accd.at[jidx], sd0).wait()

    iota16 = lax.iota(jnp.int32, 16)
    for cc in range(CHUNK // 16):
        jidx[pl.ds(cc * 16, 16)] = N_NODES + iota16

    pltpu.async_copy(ones, accd.at[jidx], sd0, add=True)

    gat(0, b0, sg0)
    gat(1, b1, sg1)

    def body(i, carry):
        k = 2 * i
        gat_wait(k, b0, sg0)
        pltpu.sync_copy(b0, acch.at[didx.at[skew + k]], add=True)
        deg_wait()
        pltpu.async_copy(ones, accd.at[didx.at[skew + k]], sd0, add=True)

        @pl.when(k + 2 < nch)
        def _():
            gat(k + 2, b0, sg0)

        gat_wait(k + 1, b1, sg1)
        pltpu.sync_copy(b1, acch.at[didx.at[skew + k + 1]], add=True)
        deg_wait()
        pltpu.async_copy(ones, accd.at[didx.at[skew + k + 1]], sd0, add=True)

        @pl.when(k + 3 < nch)
        def _():
            gat(k + 3, b1, sg1)

        return carry

    lax.fori_loop(0, nch // 2, body, 0)

    @pl.when(nch % 2 == 1)
    def _():
        last = nch - 1
        gat_wait(last, b0, sg0)
        pltpu.sync_copy(b0, acch.at[didx.at[skew + last]], add=True)
        deg_wait()
        pltpu.async_copy(ones, accd.at[didx.at[skew + last]], sd0, add=True)

    deg_wait()
    plsc.subcore_barrier()

    @pl.when(c == 0)
    def _():
        pltpu.async_copy(acch.at[myrows], h0.at[myrows], sa0)
        pltpu.async_copy(accd.at[myrows], d0.at[myrows], sd0)
        pltpu.make_async_copy(acch.at[myrows], h0.at[myrows], sa0).wait()
        pltpu.make_async_copy(accd.at[myrows], d0.at[myrows], sd0).wait()

    @pl.when(c == 1)
    def _():
        pltpu.async_copy(acch.at[myrows], h1.at[myrows], sa0)
        pltpu.async_copy(accd.at[myrows], d1.at[myrows], sd0)
        pltpu.make_async_copy(acch.at[myrows], h1.at[myrows], sa0).wait()
        pltpu.make_async_copy(accd.at[myrows], d1.at[myrows], sd0).wait()


def _combine_body(p0_ref, p1_ref, d0_ref, d1_ref, x_ref, w_ref, b_ref,
                  f_ref, ssq_ref):
    hsum = p0_ref[...] + p1_ref[...]
    deg = jnp.sum(d0_ref[...] + d1_ref[...], axis=1, keepdims=True) * (1.0 / DEGW)
    deg = jnp.maximum(deg, 1.0)
    h = hsum / deg
    w13 = w_ref[:DIM, :] + w_ref[2 * DIM:, :]
    w2 = w_ref[DIM:2 * DIM, :]
    z = jnp.dot(x_ref[...], w13, preferred_element_type=jnp.float32,
                precision=lax.Precision.HIGHEST)
    z = z + jnp.dot(h, w2, preferred_element_type=jnp.float32,
                    precision=lax.Precision.HIGHEST)
    f = jnp.tanh(z + b_ref[...])
    f_ref[...] = f

    blk = jnp.sum(f * f)[None, None]

    @pl.when(pl.program_id(0) == 0)
    def _():
        ssq_ref[...] = blk

    @pl.when(pl.program_id(0) > 0)
    def _():
        ssq_ref[...] = ssq_ref[...] + blk


def _scale_body(f_ref, ssq_ref, o_ref):
    o_ref[...] = f_ref[...] * lax.rsqrt(ssq_ref[...])


def kernel(x, edge_index, W, b):
    x = x.astype(jnp.float32)
    ei = edge_index.astype(jnp.int32).reshape(2, EROWS, CHUNK)

    zh = jnp.zeros((NPAD, DIM), jnp.float32)
    zd = jnp.zeros((NPAD, DEGW), jnp.float32)
    p0, p1, d0, d1 = _segsum_sc(x, ei, zh, zd)

    grid = N_NODES // BM
    f, ssq = pl.pallas_call(
        _combine_body,
        grid=(grid,),
        in_specs=[
            pl.BlockSpec((BM, DIM), lambda i: (i, 0)),
            pl.BlockSpec((BM, DIM), lambda i: (i, 0)),
            pl.BlockSpec((BM, DEGW), lambda i: (i, 0)),
            pl.BlockSpec((BM, DEGW), lambda i: (i, 0)),
            pl.BlockSpec((BM, DIM), lambda i: (i, 0)),
            pl.BlockSpec((3 * DIM, DIM), lambda i: (0, 0)),
            pl.BlockSpec((1, DIM), lambda i: (0, 0)),
        ],
        out_specs=[
            pl.BlockSpec((BM, DIM), lambda i: (i, 0)),
            pl.BlockSpec((1, 1), lambda i: (0, 0)),
        ],
        out_shape=[
            jax.ShapeDtypeStruct((N_NODES, DIM), jnp.float32),
            jax.ShapeDtypeStruct((1, 1), jnp.float32),
        ],
    )(p0, p1, d0, d1, x, W, b.reshape(1, DIM))

    out = pl.pallas_call(
        _scale_body,
        grid=(grid,),
        in_specs=[
            pl.BlockSpec((BM, DIM), lambda i: (i, 0)),
            pl.BlockSpec((1, 1), lambda i: (0, 0)),
        ],
        out_specs=pl.BlockSpec((BM, DIM), lambda i: (i, 0)),
        out_shape=jax.ShapeDtypeStruct((N_NODES, DIM), jnp.float32),
    )(f, ssq)
    return out
